# baseline (speedup 1.0000x reference)
import functools

import jax
import jax.numpy as jnp
from jax import lax
from jax.experimental import pallas as pl
from jax.experimental.pallas import tpu as pltpu
from jax.experimental.pallas import tpu_sc as plsc

VOCAB_SIZE = 100000
EMBED_DIM = 64
SEQ = 4096
LEN = 50
NUM_CORES = 2
NUM_SUBCORES = 16
NUM_WORKERS = NUM_CORES * NUM_SUBCORES
SPW = SEQ // NUM_WORKERS

_mesh = plsc.VectorSubcoreMesh(core_axis_name="c", subcore_axis_name="s")


@functools.partial(
    pl.kernel,
    out_type=jax.ShapeDtypeStruct((LEN, EMBED_DIM, SEQ), jnp.float32),
    mesh=_mesh,
    scratch_types=[
        pltpu.VMEM((LEN, SPW), jnp.int32),
        pltpu.VMEM((2, SPW, 128), jnp.float32),
        pltpu.VMEM((EMBED_DIM, SPW), jnp.float32),
        pltpu.SemaphoreType.DMA,
        pltpu.SemaphoreType.DMA,
    ],
    compiler_params=pltpu.CompilerParams(use_tc_tiling_on_sc=True, needs_layout_passes=False),
)
def _embed_sc(idx_hbm, table_hbm, out_hbm, idx_v, rows_v, tbuf, gsem, wsem):
    wid = lax.axis_index("s") * NUM_CORES + lax.axis_index("c")
    sb = wid * SPW
    pltpu.sync_copy(idx_hbm.at[:, pl.ds(sb, SPW)], idx_v)

    def start_gather(b, l):
        pltpu.async_copy(table_hbm.at[idx_v.at[l]], rows_v.at[b], gsem)

    def wait_gather(b, l):
        pltpu.make_async_copy(table_hbm.at[idx_v.at[l]], rows_v.at[b],
                              gsem).wait()

    lane = lax.iota(jnp.int32, 16)

    def transpose_and_write(b, l):
        for d in range(EMBED_DIM):
            col = jnp.full((16,), d, jnp.int32)
            for k in range(SPW // 16):
                v = plsc.load_gather(rows_v.at[b], [k * 16 + lane, col])
                tbuf[d, pl.ds(k * 16, 16)] = v
        pltpu.sync_copy(tbuf, out_hbm.at[l, :, pl.ds(sb, SPW)])

    start_gather(0, 0)

    def body(l, c):
        start_gather((l + 1) % 2, l + 1)
        wait_gather(l % 2, l)
        transpose_and_write(l % 2, l)
        return c

    lax.fori_loop(0, LEN - 1, body, 0)
    wait_gather((LEN - 1) % 2, LEN - 1)
    transpose_and_write((LEN - 1) % 2, LEN - 1)


def kernel(input_ids, table):
    ids_t = input_ids.astype(jnp.int32).T
    tablep = jnp.pad(table, ((0, 0), (0, 128 - EMBED_DIM)))
    out_t = _embed_sc(ids_t, tablep)
    embeds = out_t.transpose(2, 0, 1)
    return (embeds, embeds, embeds)


# speedup vs baseline: 1.6010x; 1.6010x over previous
import functools

import jax
import jax.numpy as jnp
from jax import lax
from jax.experimental import pallas as pl
from jax.experimental.pallas import tpu as pltpu
from jax.experimental.pallas import tpu_sc as plsc

VOCAB_SIZE = 100000
EMBED_DIM = 64
SEQ = 4096
LEN = 50
NUM_CORES = 2
NUM_SUBCORES = 16
NUM_WORKERS = NUM_CORES * NUM_SUBCORES
SPW = SEQ // NUM_WORKERS

_mesh = plsc.VectorSubcoreMesh(core_axis_name="c", subcore_axis_name="s")


@functools.partial(
    pl.kernel,
    out_type=jax.ShapeDtypeStruct((LEN, EMBED_DIM, SEQ), jnp.float32),
    mesh=_mesh,
    scratch_types=[
        pltpu.VMEM((LEN, SPW), jnp.int32),
        pltpu.VMEM((2, SPW, 128), jnp.float32),
        pltpu.VMEM((EMBED_DIM, SPW), jnp.float32),
        pltpu.SemaphoreType.DMA,
        pltpu.SemaphoreType.DMA,
    ],
    compiler_params=pltpu.CompilerParams(use_tc_tiling_on_sc=True, needs_layout_passes=False),
)
def _embed_sc(idx_hbm, table_hbm, out_hbm, idx_v, rows_v, tbuf, gsem, wsem):
    wid = lax.axis_index("s") * NUM_CORES + lax.axis_index("c")
    sb = wid * SPW
    pltpu.sync_copy(idx_hbm.at[:, pl.ds(sb, SPW)], idx_v)

    def start_gather(b, l):
        pltpu.async_copy(table_hbm.at[idx_v.at[l]], rows_v.at[b], gsem)

    def wait_gather(b, l):
        pltpu.make_async_copy(table_hbm.at[idx_v.at[l]], rows_v.at[b],
                              gsem).wait()

    lane = lax.iota(jnp.int32, 16)
    rots = [(lane + j) % 16 for j in range(16)]
    rvecs = [lane + r0 for r0 in range(0, SPW, 16)]

    def transpose_and_write(b, l):
        def sblock(si, c):
            rvec = lane + si * 16
            for c0 in range(0, EMBED_DIM, 16):
                for j in range(16):
                    cvec = rots[j] + c0 if c0 else rots[j]
                    v = plsc.load_gather(rows_v.at[b], [rvec, cvec])
                    plsc.store_scatter(tbuf, [cvec, rvec], v)
            return c

        lax.fori_loop(0, SPW // 16, sblock, 0)
        pltpu.sync_copy(tbuf, out_hbm.at[l, :, pl.ds(sb, SPW)])

    start_gather(0, 0)

    def body(l, c):
        start_gather((l + 1) % 2, l + 1)
        wait_gather(l % 2, l)
        transpose_and_write(l % 2, l)
        return c

    lax.fori_loop(0, LEN - 1, body, 0)
    wait_gather((LEN - 1) % 2, LEN - 1)
    transpose_and_write((LEN - 1) % 2, LEN - 1)


def kernel(input_ids, table):
    ids_t = input_ids.astype(jnp.int32).T
    tablep = jnp.pad(table, ((0, 0), (0, 128 - EMBED_DIM)))
    out_t = _embed_sc(ids_t, tablep)
    embeds = out_t.transpose(2, 0, 1)
    return (embeds, embeds, embeds)


# speedup vs baseline: 1.8140x; 1.1330x over previous
import functools

import jax
import jax.numpy as jnp
from jax import lax
from jax.experimental import pallas as pl
from jax.experimental.pallas import tpu as pltpu
from jax.experimental.pallas import tpu_sc as plsc

VOCAB_SIZE = 100000
EMBED_DIM = 64
SEQ = 4096
LEN = 50
PADW = 128
NUM_CORES = 2
NUM_SUBCORES = 16
NUM_WORKERS = NUM_CORES * NUM_SUBCORES
SPW = SEQ // NUM_WORKERS
SEQ_PER_STEP = 4
NSTEP = SPW // SEQ_PER_STEP
NBUF = 2

_mesh = plsc.VectorSubcoreMesh(core_axis_name="c", subcore_axis_name="s")


@functools.partial(
    pl.kernel,
    out_type=jax.ShapeDtypeStruct((SEQ, LEN, PADW), jnp.float32),
    mesh=_mesh,
    scratch_types=[
        pltpu.VMEM((SPW, LEN), jnp.int32),
        pltpu.VMEM((NBUF, SEQ_PER_STEP, LEN, PADW), jnp.float32),
        pltpu.SemaphoreType.DMA,
        pltpu.SemaphoreType.DMA,
    ],
    compiler_params=pltpu.CompilerParams(use_tc_tiling_on_sc=True,
                                         needs_layout_passes=False),
)
def _embed_sc(idx_hbm, table_hbm, out_hbm, idx_v, rows_v, gsem, wsem):
    wid = lax.axis_index("s") * NUM_CORES + lax.axis_index("c")
    sb = wid * SPW
    pltpu.sync_copy(idx_hbm.at[pl.ds(sb, SPW)], idx_v)

    def start_gathers(b, t):
        for si in range(SEQ_PER_STEP):
            pltpu.async_copy(table_hbm.at[idx_v.at[t * SEQ_PER_STEP + si]],
                             rows_v.at[b, si], gsem)

    def wait_gathers(b, t):
        for si in range(SEQ_PER_STEP):
            pltpu.make_async_copy(
                table_hbm.at[idx_v.at[t * SEQ_PER_STEP + si]],
                rows_v.at[b, si], gsem).wait()

    def out_slice(t):
        return out_hbm.at[pl.ds(sb + t * SEQ_PER_STEP, SEQ_PER_STEP)]

    def issue_write(b, t):
        pltpu.async_copy(rows_v.at[b], out_slice(t), wsem)

    def wait_write(b, t):
        pltpu.make_async_copy(rows_v.at[b], out_slice(t), wsem).wait()

    start_gathers(0, 0)
    start_gathers(1, 1)
    wait_gathers(0, 0)
    issue_write(0, 0)

    def body(i, c):
        for b in range(NBUF):
            t = 1 + i * NBUF + b
            bt = (b + 1) % NBUF
            bp = b
            wait_gathers(bt, t)
            issue_write(bt, t)
            wait_write(bp, t - 1)
            start_gathers(bp, t - 1 + NBUF)
        return c

    lax.fori_loop(0, (NSTEP - NBUF) // NBUF, body, 0)

    for t in range(NSTEP - NBUF + 1, NSTEP):
        wait_gathers(t % NBUF, t)
        issue_write(t % NBUF, t)
        wait_write((t - 1) % NBUF, t - 1)
    wait_write((NSTEP - 1) % NBUF, NSTEP - 1)


def kernel(input_ids, table):
    tablep = jnp.pad(table, ((0, 0), (0, PADW - EMBED_DIM)))
    out128 = _embed_sc(input_ids.astype(jnp.int32), tablep)
    embeds = out128[:, :, :EMBED_DIM]
    return (embeds, embeds, embeds)
